# baseline (speedup 1.0000x reference)
import functools

import jax
import jax.numpy as jnp
from jax import lax
from jax.experimental import pallas as pl
from jax.experimental.pallas import tpu as pltpu
from jax.experimental.pallas import tpu_sc as plsc

_NC = 2
_NS = 16
_NW = _NC * _NS
_CHUNK = 32
_NBUF = 3


@functools.partial(jax.jit, static_argnums=(2,))
def _sc_gather(table, idx, n_per_w):
  vocab, embed = table.shape
  mesh = plsc.VectorSubcoreMesh(core_axis_name="c", subcore_axis_name="s")
  n_chunks = n_per_w // _CHUNK
  n_rows = _NW * n_per_w

  @functools.partial(
      pl.kernel,
      mesh=mesh,
      out_type=jax.ShapeDtypeStruct((n_rows, embed), jnp.float32),
      scratch_types=[
          pltpu.VMEM((n_per_w,), jnp.int32),
      ] + [pltpu.VMEM((_CHUNK, embed), jnp.float32)] * _NBUF
        + [pltpu.SemaphoreType.DMA] * (2 * _NBUF),
  )
  def body(table_hbm, idx_hbm, out_hbm, idx_v, *rest):
    bufs = rest[:_NBUF]
    gsems = rest[_NBUF:2 * _NBUF]
    ssems = rest[2 * _NBUF:]
    wid = lax.axis_index("s") * _NC + lax.axis_index("c")
    base = wid * n_per_w
    seq = idx_hbm.shape[1]
    w_per_row = seq // n_per_w
    pltpu.sync_copy(
        idx_hbm.at[wid // w_per_row,
                   pl.ds((wid % w_per_row) * n_per_w, n_per_w)], idx_v)

    gathers = [None] * n_chunks
    scatters = [None] * n_chunks
    for j in range(_NBUF):
      gathers[j] = pltpu.async_copy(
          table_hbm.at[idx_v.at[pl.ds(j * _CHUNK, _CHUNK)]], bufs[j],
          gsems[j])
    for j in range(n_chunks):
      b = j % _NBUF
      gathers[j].wait()
      scatters[j] = pltpu.async_copy(
          bufs[b], out_hbm.at[pl.ds(base + j * _CHUNK, _CHUNK)], ssems[b])
      nxt = j + _NBUF
      if nxt < n_chunks:
        scatters[j].wait()
        gathers[nxt] = pltpu.async_copy(
            table_hbm.at[idx_v.at[pl.ds(nxt * _CHUNK, _CHUNK)]], bufs[b],
            gsems[b])
      else:
        scatters[j].wait()

  return body(table, idx)


def kernel(input_ids, token_embeddings):
  batch, seq = input_ids.shape
  n = batch * seq
  n_per_w = n // _NW
  out = _sc_gather(token_embeddings, input_ids.astype(jnp.int32), n_per_w)
  return out.reshape(batch, seq, token_embeddings.shape[1])


# speedup vs baseline: 1.0009x; 1.0009x over previous
import functools

import jax
import jax.numpy as jnp
from jax import lax
from jax.experimental import pallas as pl
from jax.experimental.pallas import tpu as pltpu
from jax.experimental.pallas import tpu_sc as plsc

_NC = 2
_NS = 16
_NW = _NC * _NS
_CHUNK = 32
_NBUF = 3


@functools.partial(jax.jit, static_argnums=(2,))
def _sc_gather(table, idx, n_per_w):
  embed = table.shape[1]
  mesh = plsc.VectorSubcoreMesh(core_axis_name="c", subcore_axis_name="s")
  n_chunks = n_per_w // _CHUNK
  n_rows = _NW * n_per_w

  @functools.partial(
      pl.kernel,
      mesh=mesh,
      out_type=jax.ShapeDtypeStruct((n_rows, embed), jnp.float32),
      scratch_types=[
          pltpu.VMEM((n_per_w,), jnp.int32),
      ] + [pltpu.VMEM((_CHUNK, embed), jnp.float32)] * _NBUF
        + [pltpu.SemaphoreType.DMA] * (2 * _NBUF),
  )
  def body(table_hbm, idx_hbm, out_hbm, idx_v, *rest):
    bufs = rest[:_NBUF]
    gsems = rest[_NBUF:2 * _NBUF]
    ssems = rest[2 * _NBUF:]
    wid = lax.axis_index("s") * _NC + lax.axis_index("c")
    base = wid * n_per_w
    seq = idx_hbm.shape[1]
    w_per_row = seq // n_per_w
    pltpu.sync_copy(
        idx_hbm.at[wid // w_per_row,
                   pl.ds((wid % w_per_row) * n_per_w, n_per_w)], idx_v)

    gathers = [None] * n_chunks
    scatters = [None] * n_chunks
    for j in range(_NBUF):
      gathers[j] = pltpu.async_copy(
          table_hbm.at[idx_v.at[pl.ds(j * _CHUNK, _CHUNK)]], bufs[j],
          gsems[j])
    for j in range(n_chunks):
      b = j % _NBUF
      gathers[j].wait()
      scatters[j] = pltpu.async_copy(
          bufs[b], out_hbm.at[pl.ds(base + j * _CHUNK, _CHUNK)], ssems[b])
      nxt = j + _NBUF
      if nxt < n_chunks:
        scatters[j].wait()
        gathers[nxt] = pltpu.async_copy(
            table_hbm.at[idx_v.at[pl.ds(nxt * _CHUNK, _CHUNK)]], bufs[b],
            gsems[b])
      else:
        scatters[j].wait()

  return body(table, idx)


def kernel(input_ids, token_embeddings):
  batch, seq = input_ids.shape
  n = batch * seq
  n_per_w = n // _NW
  out = _sc_gather(token_embeddings, input_ids.astype(jnp.int32), n_per_w)
  return out.reshape(batch, seq, token_embeddings.shape[1])


# speedup vs baseline: 1.0022x; 1.0013x over previous
import functools

import jax
import jax.numpy as jnp
from jax import lax
from jax.experimental import pallas as pl
from jax.experimental.pallas import tpu as pltpu
from jax.experimental.pallas import tpu_sc as plsc

_NC = 2
_NS = 16
_NW = _NC * _NS
_CHUNK = 32
_NBUF = 3


@functools.partial(jax.jit, static_argnums=(2,))
def _sc_gather(table, idx, n_per_w):
  embed = table.shape[1]
  mesh = plsc.VectorSubcoreMesh(core_axis_name="c", subcore_axis_name="s")
  n_chunks = n_per_w // _CHUNK
  n_rows = _NW * n_per_w

  @functools.partial(
      pl.kernel,
      mesh=mesh,
      out_type=jax.ShapeDtypeStruct((n_rows, embed), jnp.float32),
      scratch_types=[
          pltpu.VMEM((n_chunks, _CHUNK), jnp.int32),
      ] + [pltpu.VMEM((_CHUNK, embed), jnp.float32)] * _NBUF
        + [pltpu.SemaphoreType.DMA] * (2 * _NBUF),
  )
  def body(table_hbm, idx_hbm, out_hbm, idx_v, *rest):
    bufs = rest[:_NBUF]
    gsems = rest[_NBUF:2 * _NBUF]
    ssems = rest[2 * _NBUF:]
    wid = lax.axis_index("s") * _NC + lax.axis_index("c")
    base = wid * n_per_w
    pltpu.sync_copy(idx_hbm.at[wid], idx_v)

    gathers = [None] * n_chunks
    scatters = [None] * n_chunks
    for j in range(_NBUF):
      gathers[j] = pltpu.async_copy(
          table_hbm.at[idx_v.at[j]], bufs[j], gsems[j])
    for j in range(n_chunks):
      b = j % _NBUF
      gathers[j].wait()
      scatters[j] = pltpu.async_copy(
          bufs[b], out_hbm.at[pl.ds(base + j * _CHUNK, _CHUNK)], ssems[b])
      nxt = j + _NBUF
      if nxt < n_chunks:
        scatters[j].wait()
        gathers[nxt] = pltpu.async_copy(
            table_hbm.at[idx_v.at[nxt]], bufs[b], gsems[b])
      else:
        scatters[j].wait()

  return body(table, idx)


def kernel(input_ids, token_embeddings):
  batch, seq = input_ids.shape
  n = batch * seq
  n_per_w = n // _NW
  idx = input_ids.reshape(_NW, n_per_w // _CHUNK, _CHUNK).astype(jnp.int32)
  out = _sc_gather(token_embeddings, idx, n_per_w)
  return out.reshape(batch, seq, token_embeddings.shape[1])
